# baseline (speedup 1.0000x reference)
import functools

import jax
import jax.numpy as jnp
from jax.experimental import pallas as pl

F32 = jnp.float32

_SA_CFG = [(512, 0.1, 32), (256, 0.2, 32), (64, 0.4, 32), (16, 0.8, 32)]


def _mm_stats_kernel(x_ref, w_ref, b_ref, y_ref, s_ref):
    i = pl.program_id(0)
    y = jnp.dot(x_ref[...], w_ref[...], preferred_element_type=F32)
    if b_ref is not None:
        y = y + b_ref[...]
    y_ref[...] = y
    ssum = jnp.sum(y, axis=0, keepdims=True)
    ssq = jnp.sum(y * y, axis=0, keepdims=True)
    stat = jnp.concatenate(
        [ssum, ssq, jnp.zeros((6, y.shape[1]), F32)], axis=0)

    @pl.when(i == 0)
    def _():
        s_ref[...] = jnp.zeros_like(s_ref)

    s_ref[...] += stat


def _mm_stats(x, wt, bias=None):
    r, cin = x.shape
    cout = wt.shape[1]
    rb = min(r, 4096)
    grid = (r // rb,)
    in_specs = [
        pl.BlockSpec((rb, cin), lambda i: (i, 0)),
        pl.BlockSpec((cin, cout), lambda i: (0, 0)),
    ]
    args = [x, wt]
    if bias is not None:
        in_specs.append(pl.BlockSpec((1, cout), lambda i: (0, 0)))
        args.append(bias.reshape(1, cout))
        kern = lambda x_ref, w_ref, b_ref, y_ref, s_ref: _mm_stats_kernel(
            x_ref, w_ref, b_ref, y_ref, s_ref)
    else:
        kern = lambda x_ref, w_ref, y_ref, s_ref: _mm_stats_kernel(
            x_ref, w_ref, None, y_ref, s_ref)
    y, s = pl.pallas_call(
        kern,
        grid=grid,
        in_specs=in_specs,
        out_specs=[
            pl.BlockSpec((rb, cout), lambda i: (i, 0)),
            pl.BlockSpec((8, cout), lambda i: (0, 0)),
        ],
        out_shape=[
            jax.ShapeDtypeStruct((r, cout), F32),
            jax.ShapeDtypeStruct((8, cout), F32),
        ],
    )(*args)
    return y, s


def _norm_scale_shift(s_ref, g_ref, be_ref, rows):
    mean = s_ref[0:1, :] / rows
    var = s_ref[1:2, :] / rows - mean * mean
    scale = g_ref[...] * jax.lax.rsqrt(var + 1e-5)
    shift = be_ref[...] - mean * scale
    return scale, shift


def _nmm_kernel(y_ref, s_ref, g_ref, be_ref, w_ref, b_ref, o_ref, so_ref,
                *, rows, relu):
    i = pl.program_id(0)
    scale, shift = _norm_scale_shift(s_ref, g_ref, be_ref, rows)
    h = y_ref[...] * scale + shift
    if relu:
        h = jnp.maximum(h, 0.0)
    o = jnp.dot(h, w_ref[...], preferred_element_type=F32)
    if b_ref is not None:
        o = o + b_ref[...]
    o_ref[...] = o
    ssum = jnp.sum(o, axis=0, keepdims=True)
    ssq = jnp.sum(o * o, axis=0, keepdims=True)
    stat = jnp.concatenate(
        [ssum, ssq, jnp.zeros((6, o.shape[1]), F32)], axis=0)

    @pl.when(i == 0)
    def _():
        so_ref[...] = jnp.zeros_like(so_ref)

    so_ref[...] += stat


def _nmm(y, s, gamma, beta, wt, bias=None, relu=True):
    r, cin = y.shape
    cout = wt.shape[1]
    rb = min(r, 4096)
    grid = (r // rb,)
    in_specs = [
        pl.BlockSpec((rb, cin), lambda i: (i, 0)),
        pl.BlockSpec((8, cin), lambda i: (0, 0)),
        pl.BlockSpec((1, cin), lambda i: (0, 0)),
        pl.BlockSpec((1, cin), lambda i: (0, 0)),
        pl.BlockSpec((cin, cout), lambda i: (0, 0)),
    ]
    args = [y, s, gamma.reshape(1, cin), beta.reshape(1, cin), wt]
    if bias is not None:
        in_specs.append(pl.BlockSpec((1, cout), lambda i: (0, 0)))
        args.append(bias.reshape(1, cout))
        kern = functools.partial(_nmm_kernel, rows=float(r), relu=relu)
    else:
        def kern(y_ref, s_ref, g_ref, be_ref, w_ref, o_ref, so_ref):
            _nmm_kernel(y_ref, s_ref, g_ref, be_ref, w_ref, None, o_ref,
                        so_ref, rows=float(r), relu=relu)
    o, so = pl.pallas_call(
        kern,
        grid=grid,
        in_specs=in_specs,
        out_specs=[
            pl.BlockSpec((rb, cout), lambda i: (i, 0)),
            pl.BlockSpec((8, cout), lambda i: (0, 0)),
        ],
        out_shape=[
            jax.ShapeDtypeStruct((r, cout), F32),
            jax.ShapeDtypeStruct((8, cout), F32),
        ],
    )(*args)
    return o, so


def _norm_relu_kernel(y_ref, s_ref, g_ref, be_ref, o_ref, *, rows):
    scale, shift = _norm_scale_shift(s_ref, g_ref, be_ref, rows)
    o_ref[...] = jnp.maximum(y_ref[...] * scale + shift, 0.0)


def _norm_relu(y, s, gamma, beta):
    r, c = y.shape
    rb = min(r, 4096)
    o = pl.pallas_call(
        functools.partial(_norm_relu_kernel, rows=float(r)),
        grid=(r // rb,),
        in_specs=[
            pl.BlockSpec((rb, c), lambda i: (i, 0)),
            pl.BlockSpec((8, c), lambda i: (0, 0)),
            pl.BlockSpec((1, c), lambda i: (0, 0)),
            pl.BlockSpec((1, c), lambda i: (0, 0)),
        ],
        out_specs=pl.BlockSpec((rb, c), lambda i: (i, 0)),
        out_shape=jax.ShapeDtypeStruct((r, c), F32),
    )(y, s, gamma.reshape(1, c), beta.reshape(1, c))
    return o


def _norm_relu_pool_kernel(y_ref, s_ref, g_ref, be_ref, o_ref, *, rows):
    scale, shift = _norm_scale_shift(s_ref, g_ref, be_ref, rows)
    h = jnp.maximum(y_ref[...] * scale[None] + shift[None], 0.0)
    o_ref[...] = jnp.max(h, axis=1)


def _norm_relu_pool(y, s, gamma, beta, k):
    r, c = y.shape
    g = r // k
    gb = min(g, 1024)
    y3 = y.reshape(g, k, c)
    o = pl.pallas_call(
        functools.partial(_norm_relu_pool_kernel, rows=float(r)),
        grid=(g // gb,),
        in_specs=[
            pl.BlockSpec((gb, k, c), lambda i: (i, 0, 0)),
            pl.BlockSpec((8, c), lambda i: (0, 0)),
            pl.BlockSpec((1, c), lambda i: (0, 0)),
            pl.BlockSpec((1, c), lambda i: (0, 0)),
        ],
        out_specs=pl.BlockSpec((gb, c), lambda i: (i, 0)),
        out_shape=jax.ShapeDtypeStruct((g, c), F32),
    )(y3, s, gamma.reshape(1, c), beta.reshape(1, c))
    return o


def _mlp_chain(x, layers, pool_k=None):
    wts = [jnp.transpose(p['W']) for p in layers]
    y, s = _mm_stats(x, wts[0])
    for j in range(1, len(layers)):
        y, s = _nmm(y, s, layers[j - 1]['gamma'], layers[j - 1]['beta'],
                    wts[j])
    last = layers[-1]
    if pool_k is None:
        return _norm_relu(y, s, last['gamma'], last['beta'])
    return _norm_relu_pool(y, s, last['gamma'], last['beta'], pool_k)


def _sqdist(a, b):
    return (jnp.sum(a * a, -1)[:, :, None] + jnp.sum(b * b, -1)[:, None, :]
            - 2.0 * jnp.einsum('bmd,bnd->bmn', a, b))


def _index_points(x, idx):
    b = x.shape[0]
    bidx = jnp.arange(b).reshape((b,) + (1,) * (idx.ndim - 1))
    return x[bidx, idx]


def _fps(xyz, npoint):
    x = jax.lax.stop_gradient(xyz)
    b, n, _ = x.shape
    d0 = jnp.full((b, n), 1e10, dtype=x.dtype)
    f0 = jnp.zeros((b,), dtype=jnp.int32)

    def step(carry, _):
        dist, far = carry
        c = x[jnp.arange(b), far]
        d = jnp.sum((x - c[:, None, :]) ** 2, -1)
        dist = jnp.minimum(dist, d)
        nf = jnp.argmax(dist, axis=-1).astype(jnp.int32)
        return (dist, nf), far

    _, idxs = jax.lax.scan(step, (d0, f0), None, length=npoint)
    return jnp.transpose(idxs, (1, 0))


def _ball_query(radius, nsample, xyz, new_xyz):
    n = xyz.shape[1]
    d = jax.lax.stop_gradient(_sqdist(new_xyz, xyz))
    ar = jnp.arange(n, dtype=jnp.int32)
    gi = jnp.where(d <= radius * radius, ar[None, None, :], n)
    gi = jnp.sort(gi, axis=-1)[:, :, :nsample]
    first = gi[:, :, :1]
    return jnp.where(gi == n, jnp.broadcast_to(first, gi.shape), gi)


def _sa_module(xyz, features, npoint, radius, nsample, layers):
    b = xyz.shape[0]
    fidx = _fps(xyz, npoint)
    new_xyz = _index_points(xyz, fidx)
    gidx = _ball_query(radius, nsample, xyz, new_xyz)
    grouped_xyz = _index_points(xyz, gidx) - new_xyz[:, :, None, :]
    if features is not None:
        f = jnp.transpose(features, (0, 2, 1))
        grouped_f = _index_points(f, gidx)
        new_f = jnp.concatenate([grouped_xyz, grouped_f], axis=-1)
    else:
        new_f = grouped_xyz
    cin = new_f.shape[-1]
    x = new_f.reshape(b * npoint * nsample, cin)
    pooled = _mlp_chain(x, layers, pool_k=nsample)
    cout = pooled.shape[-1]
    new_feat = jnp.transpose(pooled.reshape(b, npoint, cout), (0, 2, 1))
    return new_xyz, new_feat


def _fp_module(unk_xyz, kn_xyz, unk_f, kn_f, layers):
    b, n, _ = unk_xyz.shape
    d = jax.lax.stop_gradient(_sqdist(unk_xyz, kn_xyz))
    negd, idx3 = jax.lax.top_k(-d, 3)
    dist3 = -negd
    w = 1.0 / (dist3 + 1e-8)
    w = w / jnp.sum(w, axis=-1, keepdims=True)
    kf = jnp.transpose(kn_f, (0, 2, 1))
    interp = jnp.sum(_index_points(kf, idx3) * w[..., None], axis=2)
    if unk_f is not None:
        x = jnp.concatenate([interp, jnp.transpose(unk_f, (0, 2, 1))], axis=-1)
    else:
        x = interp
    cin = x.shape[-1]
    h = _mlp_chain(x.reshape(b * n, cin), layers)
    cout = h.shape[-1]
    return jnp.transpose(h.reshape(b, n, cout), (0, 2, 1))


def kernel(pointcloud, params):
    pc = jnp.transpose(pointcloud, (0, 2, 1))
    xyz = pc[..., 0:3]
    features = jnp.transpose(pc[..., 3:], (0, 2, 1))
    l_xyz = [xyz]
    l_f = [features]
    for i, (npoint, radius, ns) in enumerate(_SA_CFG):
        nx, nf = _sa_module(l_xyz[i], l_f[i], npoint, radius, ns,
                            params['sa'][i])
        l_xyz.append(nx)
        l_f.append(nf)
    for i in range(-1, -5, -1):
        l_f[i - 1] = _fp_module(l_xyz[i - 1], l_xyz[i], l_f[i - 1], l_f[i],
                                params['fp'][i])

    fc = params['fc']
    b, c, n = l_f[0].shape
    x = jnp.transpose(l_f[0], (0, 2, 1)).reshape(b * n, c)
    y, s = _mm_stats(x, jnp.transpose(fc['W1']), bias=fc['b1'])
    o, _ = _nmm(y, s, fc['g1'], fc['be1'], jnp.transpose(fc['W2']),
                bias=fc['b2'], relu=False)
    return jnp.transpose(o.reshape(b, n, -1), (0, 2, 1))
